# baseline (speedup 1.0000x reference)
import jax
import jax.numpy as jnp
from jax import lax
from jax.experimental import pallas as pl
from jax.experimental.pallas import tpu as pltpu
from jax.experimental.pallas import tpu_sc as plsc

_B, _N = 1024, 8192
_L = 16
_NW = 32
_ROWS_PER_W = _B // _NW
_NCHUNKS = _N // _L


def _sc_body(x_hbm, out_hbm, row_v, out_v):
    wid = lax.axis_index("s") * 2 + lax.axis_index("c")
    iota = lax.iota(jnp.int32, _L)
    ones = jnp.ones((_L,), jnp.float32)
    zeros = jnp.zeros((_L,), jnp.float32)

    def do_row(i, carry):
        r = wid * _ROWS_PER_W + i
        pltpu.sync_copy(x_hbm.at[r], row_v)

        def red(j, cnts):
            c1, cv = cnts
            x = row_v[pl.ds(j * _L, _L)]
            c1 = c1 + plsc.all_reduce_population_count(x == 1.0)
            cv = cv + plsc.all_reduce_population_count(x != -1.0)
            return (c1, cv)

        izeros = jnp.zeros((_L,), jnp.int32)
        c1v, cvv = lax.fori_loop(0, _NCHUNKS, red, (izeros, izeros))
        c0v = cvv - c1v
        fill_v = jnp.where(cvv > 0, jnp.where(c1v > c0v, ones, zeros), ones)

        def fill_loop(j, _):
            x = row_v[pl.ds(j * _L, _L)]
            v = jnp.where(x == -1.0, fill_v, x)
            idx = j * (2 * _L) + 2 * iota
            plsc.store_scatter(out_v, [idx], ones - v)
            plsc.store_scatter(out_v, [idx + 1], v)
            return 0

        lax.fori_loop(0, _NCHUNKS, fill_loop, 0)
        pltpu.sync_copy(out_v, out_hbm.at[r])
        return carry

    lax.fori_loop(0, _ROWS_PER_W, do_row, 0)


def kernel(X):
    mesh = plsc.VectorSubcoreMesh(core_axis_name="c", subcore_axis_name="s")
    f = pl.kernel(
        _sc_body,
        mesh=mesh,
        out_type=jax.ShapeDtypeStruct((_B, 2 * _N), jnp.float32),
        scratch_types=[
            pltpu.VMEM((_N,), jnp.float32),
            pltpu.VMEM((2 * _N,), jnp.float32),
        ],
        compiler_params=pltpu.CompilerParams(needs_layout_passes=False),
    )
    out2 = f(X)
    return out2.reshape(_B, _N, 2)


# speedup vs baseline: 6.9506x; 6.9506x over previous
import jax
import jax.numpy as jnp
from jax import lax
from jax.experimental import pallas as pl
from jax.experimental.pallas import tpu as pltpu
from jax.experimental.pallas import tpu_sc as plsc

_B, _N = 1024, 8192
_L = 16
_NW = 32
_ROWS_PER_W = _B // _NW
_NCHUNKS = _N // _L
_CPB = 4
_NT = _N // 128


def _hsum(vec, scratch):
    iota = lax.iota(jnp.int32, _L)
    for sh in (1, 2, 4, 8):
        scratch[...] = vec
        vec = vec + plsc.load_gather(scratch, [iota ^ sh])
    return vec


def _sc_body(x_hbm, out_hbm, row0, row1, ob0, ob1, si0, si1, so0, so1, hs):
    wid = lax.axis_index("s") * 2 + lax.axis_index("c")
    base = wid * _ROWS_PER_W
    ones = jnp.ones((_L,), jnp.float32)
    zeros = jnp.zeros((_L,), jnp.float32)
    rows = (row0, row1)
    obufs = (ob0, ob1)
    isems = (si0, si1)
    osems = (so0, so1)

    def in_copy(r, b):
        return pltpu.make_async_copy(x_hbm.at[r], rows[b], isems[b])

    def out_copy(r, b):
        return pltpu.make_async_copy(obufs[b], out_hbm.at[r], osems[b])

    def process(i, r, b):
        buf = rows[b]
        in_copy(r, b).wait()

        @plsc.parallel_loop(0, _NCHUNKS, step=_CPB, unroll=2,
                            carry=(zeros,) * (2 * _CPB))
        def acc(j, carry):
            carry = list(carry)
            for c in range(_CPB):
                x = buf[pl.ds((j + c) * _L, _L)]
                carry[2 * c] = carry[2 * c] + x
                carry[2 * c + 1] = carry[2 * c + 1] + jnp.abs(x)
            return tuple(carry)

        s = acc[0] + acc[2] + acc[4] + acc[6]
        a = acc[1] + acc[3] + acc[5] + acc[7]
        s = _hsum(s, hs)
        a = _hsum(a, hs)
        two_n = jnp.float32(2 * _N)
        fill_v = jnp.where(a - s < two_n,
                           jnp.where(s + 3.0 * a > two_n, ones, zeros),
                           ones)

        @pl.when(i > 0)
        def _():
            out_copy(r - 2, b).wait()

        ob = obufs[b]

        @plsc.parallel_loop(0, _NT, unroll=2)
        def fill(t):
            for pos in range(128 // _L):
                x = buf[pl.ds(t * 128 + pos * _L, _L)]
                v = jnp.where(x == -1.0, fill_v, x)
                ob[2 * t, pl.ds(pos * _L, _L)] = ones - v
                ob[2 * t + 1, pl.ds(pos * _L, _L)] = v

        out_copy(r, b).start()

    in_copy(base, 0).start()

    @pl.loop(0, _ROWS_PER_W // 2)
    def pair(i):
        r = base + 2 * i
        in_copy(r + 1, 1).start()
        process(i, r, 0)

        @pl.when(i < _ROWS_PER_W // 2 - 1)
        def _():
            in_copy(r + 2, 0).start()

        process(i, r + 1, 1)

    out_copy(base + _ROWS_PER_W - 2, 0).wait()
    out_copy(base + _ROWS_PER_W - 1, 1).wait()


def kernel(X):
    mesh = plsc.VectorSubcoreMesh(core_axis_name="c", subcore_axis_name="s")
    f = pl.kernel(
        _sc_body,
        mesh=mesh,
        out_type=jax.ShapeDtypeStruct((_B, 2 * _NT, 128), jnp.float32),
        scratch_types=[
            pltpu.VMEM((_N,), jnp.float32),
            pltpu.VMEM((_N,), jnp.float32),
            pltpu.VMEM((2 * _NT, 128), jnp.float32),
            pltpu.VMEM((2 * _NT, 128), jnp.float32),
            pltpu.SemaphoreType.DMA,
            pltpu.SemaphoreType.DMA,
            pltpu.SemaphoreType.DMA,
            pltpu.SemaphoreType.DMA,
            pltpu.VMEM((_L,), jnp.float32),
        ],
        compiler_params=pltpu.CompilerParams(needs_layout_passes=False),
    )
    o = f(X)
    return (o.reshape(_B, _NT, 2, 128)
             .transpose(0, 1, 3, 2)
             .reshape(_B, _N, 2))
